# baseline (speedup 1.0000x reference)
import functools

import jax
import jax.numpy as jnp
from jax import lax
from jax.experimental import pallas as pl
from jax.experimental.pallas import tpu as pltpu
from jax.experimental.pallas import tpu_sc as plsc

NUM_LATTICES = 1024
LATTICE_RANK = 4
NUM_INPUTS = 128
BATCH = 4096
LANES = 16

NUM_CORES = 2
NUM_SUBCORES = 16
NW = NUM_CORES * NUM_SUBCORES
TB = BATCH // NW
LCHUNK = 256
NCHUNK = NUM_LATTICES // LCHUNK
GROUPS_PER_CHUNK = LCHUNK // LANES


def _tec_body(xtt_hbm, kt_hbm, idxt_hbm, out_hbm, xv, kv, iv, outv):
    wid = lax.axis_index("s") * NUM_CORES + lax.axis_index("c")

    pltpu.sync_copy(xtt_hbm.at[wid], xv)
    pltpu.sync_copy(kt_hbm, kv)
    pltpu.sync_copy(idxt_hbm, iv)

    def clip_row(r, carry):
        for j in range(NUM_INPUTS // LANES):
            v = xv[r, pl.ds(j * LANES, LANES)]
            xv[r, pl.ds(j * LANES, LANES)] = jnp.minimum(
                jnp.maximum(v, 0.0), 1.0)
        return carry

    lax.fori_loop(0, NUM_INPUTS, clip_row, 0, unroll=2)

    for c in range(NCHUNK):
        def group_body(lg, carry, c=c):
            g16 = (c * GROUPS_PER_CHUNK + lg) * LANES
            iv0 = iv[0, pl.ds(g16, LANES)]
            iv1 = iv[1, pl.ds(g16, LANES)]
            iv2 = iv[2, pl.ds(g16, LANES)]
            iv3 = iv[3, pl.ds(g16, LANES)]
            e = [kv[j, pl.ds(g16, LANES)] for j in range(8)]
            d = [kv[j + 8, pl.ds(g16, LANES)] - e[j] for j in range(8)]

            def row_body(b, rcarry, lg=lg):
                bvec = jnp.full((LANES,), b, jnp.int32)
                x0 = plsc.load_gather(xv, [iv0, bvec])
                x1 = plsc.load_gather(xv, [iv1, bvec])
                x2 = plsc.load_gather(xv, [iv2, bvec])
                x3 = plsc.load_gather(xv, [iv3, bvec])
                tA = [e[j] + d[j] * x0 for j in range(8)]
                tB = [tA[j] + (tA[j + 4] - tA[j]) * x1 for j in range(4)]
                tC = [tB[j] + (tB[j + 2] - tB[j]) * x2 for j in range(2)]
                res = tC[0] + (tC[1] - tC[0]) * x3
                outv[b, pl.ds(lg * LANES, LANES)] = res
                return rcarry

            lax.fori_loop(0, TB, row_body, 0, unroll=8)
            return carry

        lax.fori_loop(0, GROUPS_PER_CHUNK, group_body, 0)
        pltpu.sync_copy(
            outv,
            out_hbm.at[pl.ds(wid * TB, TB), pl.ds(c * LCHUNK, LCHUNK)])


@functools.partial(jax.jit, static_argnames=())
def _rtl_sc(xtt, kt, idxt):
    mesh = plsc.VectorSubcoreMesh(
        core_axis_name="c", subcore_axis_name="s")
    run = pl.kernel(
        _tec_body,
        out_type=jax.ShapeDtypeStruct((BATCH, NUM_LATTICES), jnp.float32),
        mesh=mesh,
        scratch_types=[
            pltpu.VMEM((NUM_INPUTS, TB), jnp.float32),
            pltpu.VMEM((LANES, NUM_LATTICES), jnp.float32),
            pltpu.VMEM((LATTICE_RANK, NUM_LATTICES), jnp.int32),
            pltpu.VMEM((TB, LCHUNK), jnp.float32),
        ],
        compiler_params=pltpu.CompilerParams(needs_layout_passes=False),
    )
    return run(xtt, kt, idxt)


def kernel(x, lattice_indices, kernel):
    xtt = x.reshape(NW, TB, NUM_INPUTS).transpose(0, 2, 1)
    kt = kernel.T
    idxt = lattice_indices.T.astype(jnp.int32)
    return _rtl_sc(xtt, kt, idxt)


# speedup vs baseline: 1.0374x; 1.0374x over previous
import functools

import jax
import jax.numpy as jnp
from jax import lax
from jax.experimental import pallas as pl
from jax.experimental.pallas import tpu as pltpu
from jax.experimental.pallas import tpu_sc as plsc

NUM_LATTICES = 1024
LATTICE_RANK = 4
NUM_INPUTS = 128
BATCH = 4096
LANES = 16

NUM_CORES = 2
NUM_SUBCORES = 16
NW = NUM_CORES * NUM_SUBCORES
TB = BATCH // NW
LCHUNK = 256
NCHUNK = NUM_LATTICES // LCHUNK
GROUPS_PER_CHUNK = LCHUNK // LANES


def _tec_body(xtt_hbm, kt_hbm, idxt_hbm, out_hbm, xv, kv, iv, outv):
    wid = lax.axis_index("s") * NUM_CORES + lax.axis_index("c")

    pltpu.sync_copy(xtt_hbm.at[wid], xv.at[:, pl.ds(0, TB)])
    pltpu.sync_copy(kt_hbm, kv)
    pltpu.sync_copy(idxt_hbm, iv)

    def clip_row(r, carry):
        for j in range(NUM_INPUTS // LANES):
            v = xv[r, pl.ds(j * LANES, LANES)]
            xv[r, pl.ds(j * LANES, LANES)] = jnp.minimum(
                jnp.maximum(v, 0.0), 1.0)
        return carry

    lax.fori_loop(0, NUM_INPUTS, clip_row, 0, unroll=2)

    for c in range(NCHUNK):
        def group_body(lg, carry, c=c):
            g16 = (c * GROUPS_PER_CHUNK + lg) * LANES
            iv0 = iv[0, pl.ds(g16, LANES)]
            iv1 = iv[1, pl.ds(g16, LANES)]
            iv2 = iv[2, pl.ds(g16, LANES)]
            iv3 = iv[3, pl.ds(g16, LANES)]
            e = [kv[j, pl.ds(g16, LANES)] for j in range(8)]
            d = [kv[j + 8, pl.ds(g16, LANES)] - e[j] for j in range(8)]

            def row_body(b, rcarry, lg=lg):
                bvec = jnp.full((LANES,), b, jnp.int32)
                x0 = plsc.load_gather(xv, [iv0, bvec])
                x1 = plsc.load_gather(xv, [iv1, bvec])
                x2 = plsc.load_gather(xv, [iv2, bvec])
                x3 = plsc.load_gather(xv, [iv3, bvec])
                tA = [e[j] + d[j] * x0 for j in range(8)]
                tB = [tA[j] + (tA[j + 4] - tA[j]) * x1 for j in range(4)]
                tC = [tB[j] + (tB[j + 2] - tB[j]) * x2 for j in range(2)]
                res = tC[0] + (tC[1] - tC[0]) * x3
                outv[b, pl.ds(lg * LANES, LANES)] = res
                return rcarry

            lax.fori_loop(0, TB, row_body, 0, unroll=2)
            return carry

        lax.fori_loop(0, GROUPS_PER_CHUNK, group_body, 0)
        pltpu.sync_copy(
            outv,
            out_hbm.at[pl.ds(wid * TB, TB), pl.ds(c * LCHUNK, LCHUNK)])


@functools.partial(jax.jit, static_argnames=())
def _rtl_sc(xtt, kt, idxt):
    mesh = plsc.VectorSubcoreMesh(
        core_axis_name="c", subcore_axis_name="s")
    run = pl.kernel(
        _tec_body,
        out_type=jax.ShapeDtypeStruct((BATCH, NUM_LATTICES), jnp.float32),
        mesh=mesh,
        scratch_types=[
            pltpu.VMEM((NUM_INPUTS, TB + 1), jnp.float32),
            pltpu.VMEM((LANES, NUM_LATTICES), jnp.float32),
            pltpu.VMEM((LATTICE_RANK, NUM_LATTICES), jnp.int32),
            pltpu.VMEM((TB, LCHUNK), jnp.float32),
        ],
        compiler_params=pltpu.CompilerParams(needs_layout_passes=False),
    )
    return run(xtt, kt, idxt)


def kernel(x, lattice_indices, kernel):
    xtt = x.reshape(NW, TB, NUM_INPUTS).transpose(0, 2, 1)
    kt = kernel.T
    idxt = lattice_indices.T.astype(jnp.int32)
    return _rtl_sc(xtt, kt, idxt)


# speedup vs baseline: 1.6093x; 1.5513x over previous
import functools

import jax
import jax.numpy as jnp
from jax import lax
from jax.experimental import pallas as pl
from jax.experimental.pallas import tpu as pltpu
from jax.experimental.pallas import tpu_sc as plsc

NUM_LATTICES = 1024
LATTICE_RANK = 4
NUM_INPUTS = 128
BATCH = 4096
LANES = 16

NUM_CORES = 2
NUM_SUBCORES = 16
NW = NUM_CORES * NUM_SUBCORES
TB = BATCH // NW
LCHUNK = 256
NCHUNK = NUM_LATTICES // LCHUNK
GROUPS_PER_CHUNK = LCHUNK // LANES


def _tec_body(xtt_hbm, kt_hbm, idxt_hbm, out_hbm, xv, kv, iv, outv):
    wid = lax.axis_index("s") * NUM_CORES + lax.axis_index("c")

    pltpu.sync_copy(xtt_hbm.at[wid], xv.at[:, pl.ds(0, TB)])
    pltpu.sync_copy(kt_hbm, kv)
    pltpu.sync_copy(idxt_hbm, iv)

    def clip_row(r, carry):
        for j in range(NUM_INPUTS // LANES):
            v = xv[r, pl.ds(j * LANES, LANES)]
            xv[r, pl.ds(j * LANES, LANES)] = jnp.minimum(
                jnp.maximum(v, 0.0), 1.0)
        return carry

    lax.fori_loop(0, NUM_INPUTS, clip_row, 0, unroll=2)

    for c in range(NCHUNK):
        def group_body(lg, carry, c=c):
            g16 = (c * GROUPS_PER_CHUNK + lg) * LANES
            iv0 = iv[0, pl.ds(g16, LANES)]
            iv1 = iv[1, pl.ds(g16, LANES)]
            iv2 = iv[2, pl.ds(g16, LANES)]
            iv3 = iv[3, pl.ds(g16, LANES)]
            e = [kv[j, pl.ds(g16, LANES)] for j in range(8)]
            d = [kv[j + 8, pl.ds(g16, LANES)] - e[j] for j in range(8)]

            def row_body(b, lg=lg):
                bvec = jnp.full((LANES,), b, jnp.int32)
                x0 = plsc.load_gather(xv, [iv0, bvec])
                x1 = plsc.load_gather(xv, [iv1, bvec])
                x2 = plsc.load_gather(xv, [iv2, bvec])
                x3 = plsc.load_gather(xv, [iv3, bvec])
                tA = [e[j] + d[j] * x0 for j in range(8)]
                tB = [tA[j] + (tA[j + 4] - tA[j]) * x1 for j in range(4)]
                tC = [tB[j] + (tB[j + 2] - tB[j]) * x2 for j in range(2)]
                res = tC[0] + (tC[1] - tC[0]) * x3
                outv[b, pl.ds(lg * LANES, LANES)] = res

            plsc.parallel_loop(0, TB, unroll=2)(row_body)
            return carry

        lax.fori_loop(0, GROUPS_PER_CHUNK, group_body, 0)
        pltpu.sync_copy(
            outv,
            out_hbm.at[pl.ds(wid * TB, TB), pl.ds(c * LCHUNK, LCHUNK)])


@functools.partial(jax.jit, static_argnames=())
def _rtl_sc(xtt, kt, idxt):
    mesh = plsc.VectorSubcoreMesh(
        core_axis_name="c", subcore_axis_name="s")
    run = pl.kernel(
        _tec_body,
        out_type=jax.ShapeDtypeStruct((BATCH, NUM_LATTICES), jnp.float32),
        mesh=mesh,
        scratch_types=[
            pltpu.VMEM((NUM_INPUTS, TB + 1), jnp.float32),
            pltpu.VMEM((LANES, NUM_LATTICES), jnp.float32),
            pltpu.VMEM((LATTICE_RANK, NUM_LATTICES), jnp.int32),
            pltpu.VMEM((TB, LCHUNK), jnp.float32),
        ],
        compiler_params=pltpu.CompilerParams(needs_layout_passes=False),
    )
    return run(xtt, kt, idxt)


def kernel(x, lattice_indices, kernel):
    xtt = x.reshape(NW, TB, NUM_INPUTS).transpose(0, 2, 1)
    kt = kernel.T
    idxt = lattice_indices.T.astype(jnp.int32)
    return _rtl_sc(xtt, kt, idxt)


# speedup vs baseline: 1.7452x; 1.0845x over previous
import functools

import jax
import jax.numpy as jnp
from jax import lax
from jax.experimental import pallas as pl
from jax.experimental.pallas import tpu as pltpu
from jax.experimental.pallas import tpu_sc as plsc

NUM_LATTICES = 1024
LATTICE_RANK = 4
NUM_INPUTS = 128
BATCH = 4096
LANES = 16

NUM_CORES = 2
NUM_SUBCORES = 16
NW = NUM_CORES * NUM_SUBCORES
TB = BATCH // NW
LCHUNK = 256
NCHUNK = NUM_LATTICES // LCHUNK
GROUPS_PER_CHUNK = LCHUNK // LANES


def _tec_body(xtt_hbm, kt_hbm, idxt_hbm, out_hbm, xv, kv, iv, outv):
    wid = lax.axis_index("s") * NUM_CORES + lax.axis_index("c")

    pltpu.sync_copy(xtt_hbm.at[wid], xv.at[:, pl.ds(0, TB)])
    pltpu.sync_copy(kt_hbm, kv)
    pltpu.sync_copy(idxt_hbm, iv)

    def clip_row(r, carry):
        for j in range(NUM_INPUTS // LANES):
            v = xv[r, pl.ds(j * LANES, LANES)]
            xv[r, pl.ds(j * LANES, LANES)] = jnp.minimum(
                jnp.maximum(v, 0.0), 1.0)
        return carry

    lax.fori_loop(0, NUM_INPUTS, clip_row, 0, unroll=2)

    for c in range(NCHUNK):
        def group_body(lg, carry, c=c):
            g16 = (c * GROUPS_PER_CHUNK + lg) * LANES
            iv0 = iv[0, pl.ds(g16, LANES)]
            iv1 = iv[1, pl.ds(g16, LANES)]
            iv2 = iv[2, pl.ds(g16, LANES)]
            iv3 = iv[3, pl.ds(g16, LANES)]
            e = [kv[j, pl.ds(g16, LANES)] for j in range(8)]
            d = [kv[j + 8, pl.ds(g16, LANES)] - e[j] for j in range(8)]

            def row_body(b, lg=lg):
                bvec = jnp.full((LANES,), b, jnp.int32)
                x0 = plsc.load_gather(xv, [iv0, bvec])
                x1 = plsc.load_gather(xv, [iv1, bvec])
                x2 = plsc.load_gather(xv, [iv2, bvec])
                x3 = plsc.load_gather(xv, [iv3, bvec])
                tA = [e[j] + d[j] * x0 for j in range(8)]
                tB = [tA[j] + (tA[j + 4] - tA[j]) * x1 for j in range(4)]
                tC = [tB[j] + (tB[j + 2] - tB[j]) * x2 for j in range(2)]
                res = tC[0] + (tC[1] - tC[0]) * x3
                outv[b, pl.ds(lg * LANES, LANES)] = res

            plsc.parallel_loop(0, TB, unroll=4)(row_body)
            return carry

        lax.fori_loop(0, GROUPS_PER_CHUNK, group_body, 0)
        pltpu.sync_copy(
            outv,
            out_hbm.at[pl.ds(wid * TB, TB), pl.ds(c * LCHUNK, LCHUNK)])


@functools.partial(jax.jit, static_argnames=())
def _rtl_sc(xtt, kt, idxt):
    mesh = plsc.VectorSubcoreMesh(
        core_axis_name="c", subcore_axis_name="s")
    run = pl.kernel(
        _tec_body,
        out_type=jax.ShapeDtypeStruct((BATCH, NUM_LATTICES), jnp.float32),
        mesh=mesh,
        scratch_types=[
            pltpu.VMEM((NUM_INPUTS, TB + 1), jnp.float32),
            pltpu.VMEM((LANES, NUM_LATTICES), jnp.float32),
            pltpu.VMEM((LATTICE_RANK, NUM_LATTICES), jnp.int32),
            pltpu.VMEM((TB, LCHUNK), jnp.float32),
        ],
        compiler_params=pltpu.CompilerParams(needs_layout_passes=False),
    )
    return run(xtt, kt, idxt)


def kernel(x, lattice_indices, kernel):
    xtt = x.reshape(NW, TB, NUM_INPUTS).transpose(0, 2, 1)
    kt = kernel.T
    idxt = lattice_indices.T.astype(jnp.int32)
    return _rtl_sc(xtt, kt, idxt)
